# baseline (speedup 1.0000x reference)
import jax
import jax.numpy as jnp
from jax import lax
from jax.experimental import pallas as pl
from jax.experimental.pallas import tpu as pltpu
from jax.experimental.pallas import tpu_sc as plsc

N_USERS = 100000
N_ITEMS = 100000
D = 64
L = 50
B = 4096
PAD = N_ITEMS

NC, NS = 2, 16
NW = NC * NS
BPW = B // NW
CHUNK = 4
NCHUNK = BPW // CHUNK
IDXR = CHUNK // 2
ROWS = CHUNK * L
BIAS_ROWS = (N_USERS + 1 + 127) // 128


def _sc_body(uidx_hbm, iidx_hbm, trn_hbm, ehp_hbm, etp_hbm, bup_hbm, bip_hbm,
             dt_hbm, out_hbm,
             uidx_v, iidx_v, whist_v, wbu_v, wbi_v, wtgt_v, hbuf, bbuf, tbuf,
             gidx, cnt_v, k0_v, ebuf0, ebuf1, ctx_v, bu_s, bi_s, dt_v,
             logit_v,
             sem_h, sem_t, sem_b, sem_g0, sem_g1):
    cid = lax.axis_index("c")
    sid = lax.axis_index("s")
    base = (sid * NC + cid) * BPW

    pltpu.sync_copy(uidx_hbm.at[pl.ds(base, BPW)], uidx_v)
    pltpu.sync_copy(iidx_hbm.at[pl.ds(base, BPW)], iidx_v)
    pltpu.sync_copy(dt_hbm, dt_v)

    iota = lax.iota(jnp.int32, 16)
    col = [iota + 16 * cc for cc in range(4)]

    for g in range(8):
        sl = pl.ds(g * 16, 16)
        u16 = uidx_v[sl]
        i16 = iidx_v[sl]
        whist_v[sl] = u16 >> 1
        wbu_v[sl] = u16 >> 7
        wbi_v[sl] = i16 >> 7
        wtgt_v[sl] = i16 >> 1

    hist_dma = pltpu.async_copy(trn_hbm.at[whist_v], hbuf, sem_h)
    tgt_dma = pltpu.async_copy(etp_hbm.at[wtgt_v], tbuf, sem_t)
    bu_dma = pltpu.async_copy(bup_hbm.at[wbu_v], bbuf, sem_b)

    hist_dma.wait()

    pad16 = jnp.full((16,), PAD, jnp.int32)
    one16 = jnp.full((16,), 1, jnp.int32)
    for g in range(8):
        sl = pl.ds(g * 16, 16)
        lanes = iota + g * 16
        u16 = uidx_v[sl]
        item16 = iidx_v[sl]
        off0 = (u16 & 1) * D
        grow = lanes >> 1
        gcol0 = (lanes & 1) * L

        def adj_step(l, carry, lanes=lanes, item16=item16, off0=off0,
                     grow=grow, gcol0=gcol0):
            cnt, front, back = carry
            lv = jnp.full((16,), l, jnp.int32)
            h = plsc.load_gather(hbuf, [lanes, off0 + lv])
            m = (h == item16) | (h == pad16)
            adj = jnp.where(m, pad16, h)
            even = (adj & 1) == 0
            pos = jnp.where(even, front, back)
            plsc.store_scatter(gidx, [grow, gcol0 + pos], adj >> 1)
            ev_i = even.astype(jnp.int32)
            return (cnt + m.astype(jnp.int32), front + ev_i,
                    back - (one16 - ev_i))

        def adj_body(j, carry):
            return adj_step(j * 2 + 1, adj_step(j * 2, carry))

        cnt16, k016, _ = lax.fori_loop(
            0, L // 2, adj_body,
            (jnp.zeros((16,), jnp.int32), jnp.zeros((16,), jnp.int32),
             jnp.full((16,), L - 1, jnp.int32)))
        cnt_v[sl] = cnt16
        k0_v[sl] = k016

    bu_dma.wait()
    for g in range(8):
        sl = pl.ds(g * 16, 16)
        lanes = iota + g * 16
        bu_s[sl] = plsc.load_gather(bbuf, [lanes, uidx_v[sl] & 127])
    bi_dma = pltpu.async_copy(bip_hbm.at[wbi_v], bbuf, sem_b)

    def fire(c, buf, sem):
        return [pltpu.async_copy(ehp_hbm.at[gidx.at[c * IDXR + q]],
                                 buf.at[pl.ds(q * 100, 100), :], sem)
                for q in range(IDXR)]

    def accum(c, buf):
        def elem_body(e_loc, carry, buf=buf, c=c):
            e = c * CHUNK + e_loc
            k0 = k0_v[pl.ds(e, 16)][0]

            def mk_body(off, e_loc=e_loc, buf=buf):
                def l_body(l, acc):
                    r = e_loc * L + l
                    return tuple(acc[cc] + buf[r, pl.ds(off + cc * 16, 16)]
                                 for cc in range(4))
                return l_body

            zero4 = tuple(jnp.zeros((16,), jnp.float32) for _ in range(4))
            acc = lax.fori_loop(0, k0, mk_body(0), zero4)
            acc = lax.fori_loop(k0, L, mk_body(D), acc)
            ev = jnp.full((16,), e, jnp.int32)
            for cc in range(4):
                plsc.store_scatter(ctx_v, [ev, col[cc]], acc[cc])
            return carry
        lax.fori_loop(0, CHUNK, elem_body, 0)

    bufs = (ebuf0, ebuf1)
    sems = (sem_g0, sem_g1)
    dmas = fire(0, bufs[0], sems[0])
    for c in range(NCHUNK):
        nxt = fire(c + 1, bufs[(c + 1) % 2], sems[(c + 1) % 2]) \
            if c + 1 < NCHUNK else None
        for dma in dmas:
            dma.wait()
        accum(c, bufs[c % 2])
        dmas = nxt

    bi_dma.wait()
    for g in range(8):
        sl = pl.ds(g * 16, 16)
        lanes = iota + g * 16
        bi_s[sl] = plsc.load_gather(bbuf, [lanes, iidx_v[sl] & 127])

    tgt_dma.wait()
    for g in range(8):
        sl = pl.ds(g * 16, 16)
        lanes = iota + g * 16
        denom16 = plsc.load_gather(dt_v, [cnt_v[sl]])
        toff16 = (iidx_v[sl] & 1) * D

        def dot_step(k, acc, lanes=lanes, toff16=toff16):
            kv = jnp.full((16,), k, jnp.int32)
            a = plsc.load_gather(ctx_v, [lanes, kv])
            t = plsc.load_gather(tbuf, [lanes, toff16 + kv])
            return acc + a * t

        def dot_body(j, acc):
            acc = dot_step(j * 2, acc)
            return dot_step(j * 2 + 1, acc)

        dot16 = lax.fori_loop(0, D // 2, dot_body,
                              jnp.zeros((16,), jnp.float32))
        logit_v[sl] = bu_s[sl] + bi_s[sl] + dot16 / denom16

    pltpu.sync_copy(logit_v, out_hbm.at[pl.ds(base, BPW)])


def kernel(user_idx, item_idx, trn_pos_per_user, embed_hist, embed_target,
           bias_user, bias_item):
    ehp = jnp.pad(embed_hist, ((0, 1), (0, 0))).reshape(
        (N_ITEMS + 2) // 2, 128)
    etp = jnp.pad(embed_target, ((0, 1), (0, 0))).reshape(
        (N_ITEMS + 2) // 2, 128)
    trn128 = jnp.pad(trn_pos_per_user, ((0, 0), (0, 64 - L))).reshape(
        N_USERS // 2, 128)
    bup = jnp.pad(bias_user.reshape(-1),
                  (0, BIAS_ROWS * 128 - (N_USERS + 1))).reshape(BIAS_ROWS, 128)
    bip = jnp.pad(bias_item.reshape(-1),
                  (0, BIAS_ROWS * 128 - (N_ITEMS + 1))).reshape(BIAS_ROWS, 128)
    n = jnp.arange(64, dtype=jnp.float32)
    denom_table = jnp.power(jnp.clip(n, 1e-8, None), 0.5).astype(jnp.float32)

    mesh = plsc.VectorSubcoreMesh(core_axis_name="c", subcore_axis_name="s")
    f = pl.kernel(
        _sc_body,
        out_type=jax.ShapeDtypeStruct((B,), jnp.float32),
        mesh=mesh,
        compiler_params=pltpu.CompilerParams(needs_layout_passes=False),
        scratch_types=[
            pltpu.VMEM((BPW,), jnp.int32),
            pltpu.VMEM((BPW,), jnp.int32),
            pltpu.VMEM((BPW,), jnp.int32),
            pltpu.VMEM((BPW,), jnp.int32),
            pltpu.VMEM((BPW,), jnp.int32),
            pltpu.VMEM((BPW,), jnp.int32),
            pltpu.VMEM((BPW, 128), jnp.int32),
            pltpu.VMEM((BPW, 128), jnp.float32),
            pltpu.VMEM((BPW, 128), jnp.float32),
            pltpu.VMEM((BPW // 2, 2 * L), jnp.int32),
            pltpu.VMEM((BPW,), jnp.int32),
            pltpu.VMEM((BPW + 16,), jnp.int32),
            pltpu.VMEM((ROWS, 128), jnp.float32),
            pltpu.VMEM((ROWS, 128), jnp.float32),
            pltpu.VMEM((BPW, D), jnp.float32),
            pltpu.VMEM((BPW,), jnp.float32),
            pltpu.VMEM((BPW,), jnp.float32),
            pltpu.VMEM((64,), jnp.float32),
            pltpu.VMEM((BPW,), jnp.float32),
            pltpu.SemaphoreType.DMA,
            pltpu.SemaphoreType.DMA,
            pltpu.SemaphoreType.DMA,
            pltpu.SemaphoreType.DMA,
            pltpu.SemaphoreType.DMA,
        ],
    )
    return f(user_idx, item_idx, trn128, ehp, etp, bup, bip, denom_table)


# speedup vs baseline: 1.4489x; 1.4489x over previous
import jax
import jax.numpy as jnp
from jax import lax
from jax.experimental import pallas as pl
from jax.experimental.pallas import tpu as pltpu
from jax.experimental.pallas import tpu_sc as plsc

N_USERS = 100000
N_ITEMS = 100000
D = 64
L = 50
B = 4096
PAD = N_ITEMS

NC, NS = 2, 16
NW = NC * NS
BPW = B // NW
CHUNK = 4
NCHUNK = BPW // CHUNK
IDXR = CHUNK // 2
ROWS = CHUNK * L
BIAS_ROWS = (N_USERS + 1 + 127) // 128


def _sc_body(uidx_hbm, iidx_hbm, trn_hbm, ehp_hbm, etp_hbm, bup_hbm, bip_hbm,
             dt_hbm, out_hbm,
             uidx_v, iidx_v, wbu_v, wbi_v, hbuf, bbuf, tbuf,
             gidx, cnt_v, ebuf0, ebuf1, ctx_v, bu_s, bi_s, dt_v,
             logit_v,
             sem_h, sem_t, sem_b, sem_g0, sem_g1):
    cid = lax.axis_index("c")
    sid = lax.axis_index("s")
    base = (sid * NC + cid) * BPW

    pltpu.sync_copy(uidx_hbm.at[pl.ds(base, BPW)], uidx_v)
    pltpu.sync_copy(iidx_hbm.at[pl.ds(base, BPW)], iidx_v)
    pltpu.sync_copy(dt_hbm, dt_v)

    iota = lax.iota(jnp.int32, 16)
    col = [iota + 16 * cc for cc in range(4)]

    for g in range(8):
        sl = pl.ds(g * 16, 16)
        u16 = uidx_v[sl]
        i16 = iidx_v[sl]
        wbu_v[sl] = u16 >> 7
        wbi_v[sl] = i16 >> 7

    hist_dma = pltpu.async_copy(trn_hbm.at[uidx_v], hbuf, sem_h)
    tgt_dma = pltpu.async_copy(etp_hbm.at[iidx_v], tbuf, sem_t)
    bu_dma = pltpu.async_copy(bup_hbm.at[wbu_v], bbuf, sem_b)

    hist_dma.wait()

    pad16 = jnp.full((16,), PAD, jnp.int32)
    for g in range(8):
        sl = pl.ds(g * 16, 16)
        lanes = iota + g * 16
        item16 = iidx_v[sl]
        grow = lanes >> 1
        gcol0 = (lanes & 1) * L

        def adj_step(l, cnt, lanes=lanes, item16=item16,
                     grow=grow, gcol0=gcol0):
            lv = jnp.full((16,), l, jnp.int32)
            h = plsc.load_gather(hbuf, [lanes, lv])
            m = (h == item16) | (h == pad16)
            adj = jnp.where(m, pad16, h)
            plsc.store_scatter(gidx, [grow, gcol0 + lv], adj)
            return cnt + m.astype(jnp.int32)

        def adj_body(j, cnt):
            return adj_step(j * 2 + 1, adj_step(j * 2, cnt))

        cnt16 = lax.fori_loop(0, L // 2, adj_body,
                              jnp.zeros((16,), jnp.int32))
        cnt_v[sl] = cnt16

    bu_dma.wait()
    for g in range(8):
        sl = pl.ds(g * 16, 16)
        lanes = iota + g * 16
        bu_s[sl] = plsc.load_gather(bbuf, [lanes, uidx_v[sl] & 127])
    bi_dma = pltpu.async_copy(bip_hbm.at[wbi_v], bbuf, sem_b)

    def fire(c, buf, sem):
        return [pltpu.async_copy(ehp_hbm.at[gidx.at[c * IDXR + q]],
                                 buf.at[pl.ds(q * 100, 100), :], sem)
                for q in range(IDXR)]

    def accum(c, buf):
        def elem_body(e_loc, carry, buf=buf, c=c):
            e = c * CHUNK + e_loc

            def l_body(j, acc, e_loc=e_loc, buf=buf):
                r = e_loc * L + j * 2
                a = tuple(acc[cc] + buf[r, pl.ds(cc * 16, 16)]
                          for cc in range(4))
                return tuple(a[cc] + buf[r + 1, pl.ds(cc * 16, 16)]
                             for cc in range(4))

            acc = lax.fori_loop(
                0, L // 2, l_body,
                tuple(jnp.zeros((16,), jnp.float32) for _ in range(4)))
            ev = jnp.full((16,), e, jnp.int32)
            for cc in range(4):
                plsc.store_scatter(ctx_v, [ev, col[cc]], acc[cc])
            return carry
        lax.fori_loop(0, CHUNK, elem_body, 0)

    bufs = (ebuf0, ebuf1)
    sems = (sem_g0, sem_g1)
    dmas = fire(0, bufs[0], sems[0])
    for c in range(NCHUNK):
        nxt = fire(c + 1, bufs[(c + 1) % 2], sems[(c + 1) % 2]) \
            if c + 1 < NCHUNK else None
        for dma in dmas:
            dma.wait()
        accum(c, bufs[c % 2])
        dmas = nxt

    bi_dma.wait()
    for g in range(8):
        sl = pl.ds(g * 16, 16)
        lanes = iota + g * 16
        bi_s[sl] = plsc.load_gather(bbuf, [lanes, iidx_v[sl] & 127])

    tgt_dma.wait()
    for g in range(8):
        sl = pl.ds(g * 16, 16)
        lanes = iota + g * 16
        denom16 = plsc.load_gather(dt_v, [cnt_v[sl]])

        def dot_step(k, acc, lanes=lanes):
            kv = jnp.full((16,), k, jnp.int32)
            a = plsc.load_gather(ctx_v, [lanes, kv])
            t = plsc.load_gather(tbuf, [lanes, kv])
            return acc + a * t

        def dot_body(j, acc):
            acc = dot_step(j * 2, acc)
            return dot_step(j * 2 + 1, acc)

        dot16 = lax.fori_loop(0, D // 2, dot_body,
                              jnp.zeros((16,), jnp.float32))
        logit_v[sl] = bu_s[sl] + bi_s[sl] + dot16 / denom16

    pltpu.sync_copy(logit_v, out_hbm.at[pl.ds(base, BPW)])


def kernel(user_idx, item_idx, trn_pos_per_user, embed_hist, embed_target,
           bias_user, bias_item):
    ehp = jnp.pad(embed_hist, ((0, 0), (0, 128 - D)))
    etp = jnp.pad(embed_target, ((0, 0), (0, 128 - D)))
    trn128 = jnp.pad(trn_pos_per_user, ((0, 0), (0, 128 - L)))
    bup = jnp.pad(bias_user.reshape(-1),
                  (0, BIAS_ROWS * 128 - (N_USERS + 1))).reshape(BIAS_ROWS, 128)
    bip = jnp.pad(bias_item.reshape(-1),
                  (0, BIAS_ROWS * 128 - (N_ITEMS + 1))).reshape(BIAS_ROWS, 128)
    n = jnp.arange(64, dtype=jnp.float32)
    denom_table = jnp.power(jnp.clip(n, 1e-8, None), 0.5).astype(jnp.float32)

    mesh = plsc.VectorSubcoreMesh(core_axis_name="c", subcore_axis_name="s")
    f = pl.kernel(
        _sc_body,
        out_type=jax.ShapeDtypeStruct((B,), jnp.float32),
        mesh=mesh,
        compiler_params=pltpu.CompilerParams(needs_layout_passes=False),
        scratch_types=[
            pltpu.VMEM((BPW,), jnp.int32),
            pltpu.VMEM((BPW,), jnp.int32),
            pltpu.VMEM((BPW,), jnp.int32),
            pltpu.VMEM((BPW,), jnp.int32),
            pltpu.VMEM((BPW, 128), jnp.int32),
            pltpu.VMEM((BPW, 128), jnp.float32),
            pltpu.VMEM((BPW, 128), jnp.float32),
            pltpu.VMEM((BPW // 2, 2 * L), jnp.int32),
            pltpu.VMEM((BPW,), jnp.int32),
            pltpu.VMEM((ROWS, 128), jnp.float32),
            pltpu.VMEM((ROWS, 128), jnp.float32),
            pltpu.VMEM((BPW, D), jnp.float32),
            pltpu.VMEM((BPW,), jnp.float32),
            pltpu.VMEM((BPW,), jnp.float32),
            pltpu.VMEM((64,), jnp.float32),
            pltpu.VMEM((BPW,), jnp.float32),
            pltpu.SemaphoreType.DMA,
            pltpu.SemaphoreType.DMA,
            pltpu.SemaphoreType.DMA,
            pltpu.SemaphoreType.DMA,
            pltpu.SemaphoreType.DMA,
        ],
    )
    return f(user_idx, item_idx, trn128, ehp, etp, bup, bip, denom_table)
